# Initial kernel scaffold; baseline (speedup 1.0000x reference)
import jax
import jax.numpy as jnp
from jax.experimental import pallas as pl


def kernel(encoded_words, embed_weight):
    raise NotImplementedError("write your pallas kernel here")


# speedup vs baseline: 6.0326x; 6.0326x over previous
import functools
import math

import jax
import jax.numpy as jnp
from jax import lax
from jax.experimental import pallas as pl
from jax.experimental.pallas import tpu as pltpu
from jax.experimental.pallas import tpu_sc as plsc

DIM = 128
SEQ = 200
BATCH = 1024
N_TOKENS = BATCH * SEQ
SCALE = math.sqrt(float(DIM))

NW = 32
PER_W = N_TOKENS // NW
CHUNK = SEQ
N_CHUNKS = PER_W // CHUNK
SPLIT_A = 96
SPLIT_B = CHUNK - SPLIT_A
LANES = 16


def _pos_encoding():
    pos = jnp.arange(SEQ, dtype=jnp.float32)[:, None]
    i = jnp.arange(DIM, dtype=jnp.float32)[None, :]
    div = jnp.power(10000.0, 2.0 * i / DIM)
    angles = pos / div
    is_odd = (jnp.arange(DIM) % 2 == 1)[None, :]
    return jnp.where(is_odd, jnp.cos(angles), jnp.sin(angles))


_mesh = plsc.VectorSubcoreMesh(core_axis_name="c", subcore_axis_name="s")


@functools.partial(
    pl.kernel,
    mesh=_mesh,
    out_type=jax.ShapeDtypeStruct((N_TOKENS, DIM), jnp.float32),
    scratch_types=[
        pltpu.VMEM((SPLIT_A,), jnp.int32),
        pltpu.VMEM((SPLIT_B,), jnp.int32),
        pltpu.VMEM((SPLIT_A,), jnp.int32),
        pltpu.VMEM((SPLIT_B,), jnp.int32),
        pltpu.VMEM((CHUNK, DIM), jnp.float32),
        pltpu.VMEM((CHUNK, DIM), jnp.float32),
        pltpu.VMEM((SEQ, DIM), jnp.float32),
        pltpu.SemaphoreType.DMA,
        pltpu.SemaphoreType.DMA,
        pltpu.SemaphoreType.DMA,
        pltpu.SemaphoreType.DMA,
    ],
)
def _sc_embed(table, idx, pe, out,
              ia0, ib0, ia1, ib1, buf0, buf1, pe_v, g0, g1, o0, o1):
    wid = lax.axis_index("s") * 2 + lax.axis_index("c")
    base = wid * PER_W

    idx_a = (ia0, ia1)
    idx_b = (ib0, ib1)
    bufs = (buf0, buf1)
    gsems = (g0, g1)
    osems = (o0, o1)

    pltpu.sync_copy(pe, pe_v)

    def load_idx(b, c):
        off = base + c * CHUNK
        pltpu.sync_copy(idx.at[pl.ds(off, SPLIT_A)], idx_a[b])
        pltpu.sync_copy(idx.at[pl.ds(off + SPLIT_A, SPLIT_B)], idx_b[b])

    def start_gather(b):
        pltpu.async_copy(table.at[idx_a[b]], bufs[b].at[pl.ds(0, SPLIT_A)],
                         gsems[b])
        pltpu.async_copy(table.at[idx_b[b]], bufs[b].at[pl.ds(SPLIT_A, SPLIT_B)],
                         gsems[b])

    def wait_gather(b):
        pltpu.make_async_copy(table.at[pl.ds(0, CHUNK)], bufs[b],
                              gsems[b]).wait()

    def compute(b):
        buf = bufs[b]

        def row(r, carry):
            for k in range(DIM // LANES):
                sl = pl.ds(k * LANES, LANES)
                buf[r, sl] = buf[r, sl] * SCALE + pe_v[r, sl]
            return carry

        lax.fori_loop(0, CHUNK, row, 0)

    def start_out(b, c):
        off = base + c * CHUNK
        pltpu.async_copy(bufs[b], out.at[pl.ds(off, CHUNK)], osems[b])

    def wait_out(b):
        pltpu.make_async_copy(bufs[b], out.at[pl.ds(0, CHUNK)],
                              osems[b]).wait()

    for b in range(2):
        load_idx(b, b)
        start_gather(b)

    def main(i, carry):
        for b in range(2):
            c = i * 2 + b
            wait_gather(b)
            compute(b)
            start_out(b, c)
            load_idx(b, c + 2)
            wait_out(b)
            start_gather(b)
        return carry

    lax.fori_loop(0, (N_CHUNKS - 2) // 2, main, 0)

    for b in range(2):
        c = N_CHUNKS - 2 + b
        wait_gather(b)
        compute(b)
        start_out(b, c)
        wait_out(b)


def kernel(encoded_words, embed_weight):
    idx = encoded_words.reshape(-1).astype(jnp.int32)
    pe = _pos_encoding()
    out = _sc_embed(embed_weight, idx, pe)
    return out.reshape(encoded_words.shape[0], encoded_words.shape[1], DIM)


# speedup vs baseline: 7.2919x; 1.2087x over previous
import functools
import math

import jax
import jax.numpy as jnp
from jax import lax
from jax.experimental import pallas as pl
from jax.experimental.pallas import tpu as pltpu
from jax.experimental.pallas import tpu_sc as plsc

DIM = 128
SEQ = 200
BATCH = 1024
N_TOKENS = BATCH * SEQ
SCALE = math.sqrt(float(DIM))

NW = 32
PER_W = N_TOKENS // NW
CHUNK = SEQ
N_CHUNKS = PER_W // CHUNK
SPLIT_A = 96
SPLIT_B = CHUNK - SPLIT_A
LANES = 16
NBUF = 3


def _pos_encoding():
    pos = jnp.arange(SEQ, dtype=jnp.float32)[:, None]
    i = jnp.arange(DIM, dtype=jnp.float32)[None, :]
    div = jnp.power(10000.0, 2.0 * i / DIM)
    angles = pos / div
    is_odd = (jnp.arange(DIM) % 2 == 1)[None, :]
    return jnp.where(is_odd, jnp.cos(angles), jnp.sin(angles))


_mesh = plsc.VectorSubcoreMesh(core_axis_name="c", subcore_axis_name="s")


@functools.partial(
    pl.kernel,
    mesh=_mesh,
    out_type=jax.ShapeDtypeStruct((N_TOKENS, DIM), jnp.float32),
    scratch_types=[
        pltpu.VMEM((PER_W,), jnp.int32),
        pltpu.VMEM((CHUNK, DIM), jnp.float32),
        pltpu.VMEM((CHUNK, DIM), jnp.float32),
        pltpu.VMEM((CHUNK, DIM), jnp.float32),
        pltpu.VMEM((SEQ, DIM), jnp.float32),
        pltpu.SemaphoreType.DMA,
        pltpu.SemaphoreType.DMA,
        pltpu.SemaphoreType.DMA,
        pltpu.SemaphoreType.DMA,
        pltpu.SemaphoreType.DMA,
        pltpu.SemaphoreType.DMA,
    ],
)
def _sc_embed(table, idx, pe, out,
              idx_v, buf0, buf1, buf2, pe_v, g0, g1, g2, o0, o1, o2):
    wid = lax.axis_index("s") * 2 + lax.axis_index("c")
    base = wid * PER_W

    bufs = (buf0, buf1, buf2)
    gsems = (g0, g1, g2)
    osems = (o0, o1, o2)

    pltpu.sync_copy(idx.at[pl.ds(base, PER_W)], idx_v)
    pltpu.sync_copy(pe, pe_v)

    def start_gather(c, b):
        off = c * CHUNK
        pltpu.async_copy(table.at[idx_v.at[pl.ds(off, SPLIT_A)]],
                         bufs[b].at[pl.ds(0, SPLIT_A)], gsems[b])
        pltpu.async_copy(table.at[idx_v.at[pl.ds(off + SPLIT_A, SPLIT_B)]],
                         bufs[b].at[pl.ds(SPLIT_A, SPLIT_B)], gsems[b])

    def wait_gather(b):
        pltpu.make_async_copy(table.at[pl.ds(0, CHUNK)], bufs[b],
                              gsems[b]).wait()

    def compute(b):
        buf = bufs[b]

        def row(r, carry):
            for k in range(DIM // LANES):
                sl = pl.ds(k * LANES, LANES)
                buf[r, sl] = buf[r, sl] * SCALE + pe_v[r, sl]
            return carry

        lax.fori_loop(0, CHUNK, row, 0)

    def start_out(c, b):
        off = base + c * CHUNK
        pltpu.async_copy(bufs[b], out.at[pl.ds(off, CHUNK)], osems[b])

    def wait_out(b):
        pltpu.make_async_copy(bufs[b], out.at[pl.ds(0, CHUNK)],
                              osems[b]).wait()

    def step(c, b, *, first_use_next=False, start_next=True):
        wait_gather(b)
        compute(b)
        start_out(c, b)
        if start_next:
            bn = (b + 2) % NBUF
            if not first_use_next:
                wait_out(bn)
            start_gather(c + 2, bn)

    start_gather(0, 0)
    start_gather(1, 1)

    step(0, 0, first_use_next=True)

    def main(i, carry):
        for j in range(NBUF):
            c = 1 + i * NBUF + j
            step(c, (1 + j) % NBUF)
        return carry

    lax.fori_loop(0, 9, main, 0)

    step(28, (28 % NBUF))
    step(29, (29 % NBUF))
    step(30, (30 % NBUF), start_next=False)
    step(31, (31 % NBUF), start_next=False)

    for b in range(NBUF):
        wait_out(b)


def kernel(encoded_words, embed_weight):
    idx = encoded_words.reshape(-1).astype(jnp.int32)
    pe = _pos_encoding()
    out = _sc_embed(embed_weight, idx, pe)
    return out.reshape(encoded_words.shape[0], encoded_words.shape[1], DIM)


# speedup vs baseline: 7.3691x; 1.0106x over previous
import functools
import math

import jax
import jax.numpy as jnp
from jax import lax
from jax.experimental import pallas as pl
from jax.experimental.pallas import tpu as pltpu
from jax.experimental.pallas import tpu_sc as plsc

DIM = 128
SEQ = 200
BATCH = 1024
N_TOKENS = BATCH * SEQ
SCALE = math.sqrt(float(DIM))

NW = 32
PER_W = N_TOKENS // NW
CHUNK = SEQ
N_CHUNKS = PER_W // CHUNK
SPLIT_A = 96
SPLIT_B = CHUNK - SPLIT_A
LANES = 16
NBUF = 3


def _pos_encoding():
    pos = jnp.arange(SEQ, dtype=jnp.float32)[:, None]
    i = jnp.arange(DIM, dtype=jnp.float32)[None, :]
    div = jnp.power(10000.0, 2.0 * i / DIM)
    angles = pos / div
    is_odd = (jnp.arange(DIM) % 2 == 1)[None, :]
    return jnp.where(is_odd, jnp.cos(angles), jnp.sin(angles))


_mesh = plsc.VectorSubcoreMesh(core_axis_name="c", subcore_axis_name="s")


@functools.partial(
    pl.kernel,
    mesh=_mesh,
    out_type=jax.ShapeDtypeStruct((N_TOKENS, DIM), jnp.float32),
    scratch_types=[
        pltpu.VMEM((PER_W,), jnp.int32),
        pltpu.VMEM((CHUNK, DIM), jnp.float32),
        pltpu.VMEM((CHUNK, DIM), jnp.float32),
        pltpu.VMEM((CHUNK, DIM), jnp.float32),
        pltpu.VMEM((SEQ, DIM), jnp.float32),
        pltpu.SemaphoreType.DMA,
        pltpu.SemaphoreType.DMA,
        pltpu.SemaphoreType.DMA,
        pltpu.SemaphoreType.DMA,
        pltpu.SemaphoreType.DMA,
        pltpu.SemaphoreType.DMA,
        pltpu.SemaphoreType.DMA,
    ],
)
def _sc_embed(table, idx, pe, out,
              idx_v, buf0, buf1, buf2, pe_v, g0, g1, g2, o0, o1, o2, psem):
    wid = lax.axis_index("s") * 2 + lax.axis_index("c")
    base = wid * PER_W

    bufs = (buf0, buf1, buf2)
    gsems = (g0, g1, g2)
    osems = (o0, o1, o2)

    pltpu.async_copy(pe, pe_v, psem)
    pltpu.sync_copy(idx.at[pl.ds(base, PER_W)], idx_v)

    def start_gather(c, b):
        off = c * CHUNK
        pltpu.async_copy(table.at[idx_v.at[pl.ds(off, SPLIT_A)]],
                         bufs[b].at[pl.ds(0, SPLIT_A)], gsems[b])
        pltpu.async_copy(table.at[idx_v.at[pl.ds(off + SPLIT_A, SPLIT_B)]],
                         bufs[b].at[pl.ds(SPLIT_A, SPLIT_B)], gsems[b])

    def wait_gather(b):
        pltpu.make_async_copy(table.at[pl.ds(0, CHUNK)], bufs[b],
                              gsems[b]).wait()

    def compute(b):
        buf = bufs[b]

        def row(r, carry):
            for k in range(DIM // LANES):
                sl = pl.ds(k * LANES, LANES)
                buf[r, sl] = buf[r, sl] * SCALE + pe_v[r, sl]
            return carry

        lax.fori_loop(0, CHUNK, row, 0)

    def start_out(c, b):
        off = base + c * CHUNK
        pltpu.async_copy(bufs[b], out.at[pl.ds(off, CHUNK)], osems[b])

    def wait_out(b):
        pltpu.make_async_copy(bufs[b], out.at[pl.ds(0, CHUNK)],
                              osems[b]).wait()

    def step(c, b, *, first_use_next=False, start_next=True):
        wait_gather(b)
        compute(b)
        start_out(c, b)
        if start_next:
            bn = (b + 2) % NBUF
            if not first_use_next:
                wait_out(bn)
            start_gather(c + 2, bn)

    start_gather(0, 0)
    start_gather(1, 1)
    pltpu.make_async_copy(pe, pe_v, psem).wait()

    step(0, 0, first_use_next=True)

    def main(i, carry):
        for j in range(NBUF):
            c = 1 + i * NBUF + j
            step(c, (1 + j) % NBUF)
        return carry

    lax.fori_loop(0, 9, main, 0)

    step(28, (28 % NBUF))
    step(29, (29 % NBUF))
    step(30, (30 % NBUF), start_next=False)
    step(31, (31 % NBUF), start_next=False)

    for b in range(NBUF):
        wait_out(b)


def kernel(encoded_words, embed_weight):
    idx = encoded_words.reshape(-1).astype(jnp.int32)
    pe = _pos_encoding()
    out = _sc_embed(embed_weight, idx, pe)
    return out.reshape(encoded_words.shape[0], encoded_words.shape[1], DIM)


# speedup vs baseline: 7.3787x; 1.0013x over previous
import functools
import math

import jax
import jax.numpy as jnp
from jax import lax
from jax.experimental import pallas as pl
from jax.experimental.pallas import tpu as pltpu
from jax.experimental.pallas import tpu_sc as plsc

DIM = 128
SEQ = 200
BATCH = 1024
N_TOKENS = BATCH * SEQ
SCALE = math.sqrt(float(DIM))

NW = 32
PER_W = N_TOKENS // NW
CHUNK = SEQ
N_CHUNKS = PER_W // CHUNK
SPLIT_A = 96
SPLIT_B = CHUNK - SPLIT_A
LANES = 16
NBUF = 4
AHEAD = 3


def _pos_encoding():
    pos = jnp.arange(SEQ, dtype=jnp.float32)[:, None]
    i = jnp.arange(DIM, dtype=jnp.float32)[None, :]
    div = jnp.power(10000.0, 2.0 * i / DIM)
    angles = pos / div
    is_odd = (jnp.arange(DIM) % 2 == 1)[None, :]
    return jnp.where(is_odd, jnp.cos(angles), jnp.sin(angles))


_mesh = plsc.VectorSubcoreMesh(core_axis_name="c", subcore_axis_name="s")


@functools.partial(
    pl.kernel,
    mesh=_mesh,
    out_type=jax.ShapeDtypeStruct((N_TOKENS, DIM), jnp.float32),
    scratch_types=[pltpu.VMEM((CHUNK, DIM), jnp.float32)] * NBUF
    + [pltpu.VMEM((SEQ, DIM), jnp.float32)]
    + [pltpu.VMEM((SPLIT_A,), jnp.int32), pltpu.VMEM((SPLIT_B,), jnp.int32)] * NBUF
    + [pltpu.SemaphoreType.DMA] * (3 * NBUF + 1),
)
def _sc_embed(table, idx, pe, out, *refs):
    bufs = refs[:NBUF]
    pe_v = refs[NBUF]
    ia = refs[NBUF + 1:NBUF + 1 + 2 * NBUF:2]
    ib = refs[NBUF + 2:NBUF + 2 + 2 * NBUF:2]
    gsems = refs[3 * NBUF + 1:4 * NBUF + 1]
    osems = refs[4 * NBUF + 1:5 * NBUF + 1]
    isems = refs[5 * NBUF + 1:6 * NBUF + 1]
    psem = refs[6 * NBUF + 1]

    wid = lax.axis_index("s") * 2 + lax.axis_index("c")
    base = wid * PER_W

    pltpu.async_copy(pe, pe_v, psem)

    def start_idx(c, b):
        off = base + c * CHUNK
        pltpu.async_copy(idx.at[pl.ds(off, SPLIT_A)], ia[b], isems[b])
        pltpu.async_copy(idx.at[pl.ds(off + SPLIT_A, SPLIT_B)], ib[b],
                         isems[b])

    def wait_idx(b):
        pltpu.make_async_copy(idx.at[pl.ds(0, SPLIT_A)], ia[b],
                              isems[b]).wait()
        pltpu.make_async_copy(idx.at[pl.ds(0, SPLIT_B)], ib[b],
                              isems[b]).wait()

    def start_gather(c, b):
        pltpu.async_copy(table.at[ia[b]], bufs[b].at[pl.ds(0, SPLIT_A)],
                         gsems[b])
        pltpu.async_copy(table.at[ib[b]], bufs[b].at[pl.ds(SPLIT_A, SPLIT_B)],
                         gsems[b])

    def wait_gather(b):
        pltpu.make_async_copy(table.at[pl.ds(0, CHUNK)], bufs[b],
                              gsems[b]).wait()

    def compute(b):
        buf = bufs[b]

        def row(r, carry):
            for k in range(DIM // LANES):
                sl = pl.ds(k * LANES, LANES)
                buf[r, sl] = buf[r, sl] * SCALE + pe_v[r, sl]
            return carry

        lax.fori_loop(0, CHUNK, row, 0)

    def start_out(c, b):
        off = base + c * CHUNK
        pltpu.async_copy(bufs[b], out.at[pl.ds(off, CHUNK)], osems[b])

    def wait_out(b):
        pltpu.make_async_copy(bufs[b], out.at[pl.ds(0, CHUNK)],
                              osems[b]).wait()

    def step(c, b, *, drain_next=True, load_next_idx=True, start_next=True):
        wait_gather(b)
        compute(b)
        start_out(c, b)
        if load_next_idx:
            start_idx(c + NBUF, b)
        if start_next:
            bn = (b + AHEAD) % NBUF
            if drain_next:
                wait_out(bn)
            wait_idx(bn)
            start_gather(c + AHEAD, bn)

    for c in range(NBUF):
        start_idx(c, c)
    for c in range(AHEAD):
        wait_idx(c)
        start_gather(c, c)
    pltpu.make_async_copy(pe, pe_v, psem).wait()

    step(0, 0, drain_next=False)

    def main(i, carry):
        for j in range(NBUF):
            c = 1 + i * NBUF + j
            step(c, (1 + j) % NBUF)
        return carry

    lax.fori_loop(0, 6, main, 0)

    step(25, 25 % NBUF)
    step(26, 26 % NBUF)
    step(27, 27 % NBUF)
    step(28, 28 % NBUF, load_next_idx=False)
    for c in range(29, 32):
        step(c, c % NBUF, load_next_idx=False, start_next=False)

    for b in range(NBUF):
        wait_out(b)


def kernel(encoded_words, embed_weight):
    idx = encoded_words.reshape(-1).astype(jnp.int32)
    pe = _pos_encoding()
    out = _sc_embed(embed_weight, idx, pe)
    return out.reshape(encoded_words.shape[0], encoded_words.shape[1], DIM)


# speedup vs baseline: 7.3788x; 1.0000x over previous
import functools
import math

import jax
import jax.numpy as jnp
from jax import lax
from jax.experimental import pallas as pl
from jax.experimental.pallas import tpu as pltpu
from jax.experimental.pallas import tpu_sc as plsc

DIM = 128
SEQ = 200
BATCH = 1024
N_TOKENS = BATCH * SEQ
SCALE = math.sqrt(float(DIM))

NW = 32
PER_W = N_TOKENS // NW
CHUNK = SEQ
N_CHUNKS = PER_W // CHUNK
SPLIT_A = 96
SPLIT_B = CHUNK - SPLIT_A
LANES = 16
NBUF = 3


def _pos_encoding():
    pos = jnp.arange(SEQ, dtype=jnp.float32)[:, None]
    i = jnp.arange(DIM, dtype=jnp.float32)[None, :]
    div = jnp.power(10000.0, 2.0 * i / DIM)
    angles = pos / div
    is_odd = (jnp.arange(DIM) % 2 == 1)[None, :]
    return jnp.where(is_odd, jnp.cos(angles), jnp.sin(angles))


_mesh = plsc.VectorSubcoreMesh(core_axis_name="c", subcore_axis_name="s")


@functools.partial(
    pl.kernel,
    mesh=_mesh,
    out_type=jax.ShapeDtypeStruct((N_TOKENS, DIM), jnp.float32),
    scratch_types=[
        pltpu.VMEM((PER_W,), jnp.int32),
        pltpu.VMEM((CHUNK, DIM), jnp.float32),
        pltpu.VMEM((CHUNK, DIM), jnp.float32),
        pltpu.VMEM((CHUNK, DIM), jnp.float32),
        pltpu.VMEM((SEQ, DIM), jnp.float32),
        pltpu.SemaphoreType.DMA,
        pltpu.SemaphoreType.DMA,
        pltpu.SemaphoreType.DMA,
        pltpu.SemaphoreType.DMA,
        pltpu.SemaphoreType.DMA,
        pltpu.SemaphoreType.DMA,
        pltpu.SemaphoreType.DMA,
    ],
)
def _sc_embed(table, idx, pe, out,
              idx_v, buf0, buf1, buf2, pe_v, g0, g1, g2, o0, o1, o2, psem):
    wid = lax.axis_index("s") * 2 + lax.axis_index("c")
    base = wid * PER_W

    bufs = (buf0, buf1, buf2)
    gsems = (g0, g1, g2)
    osems = (o0, o1, o2)

    pltpu.async_copy(pe, pe_v, psem)
    pltpu.sync_copy(idx.at[pl.ds(base, PER_W)], idx_v)

    def start_gather(c, b):
        off = c * CHUNK
        pltpu.async_copy(table.at[idx_v.at[pl.ds(off, SPLIT_A)]],
                         bufs[b].at[pl.ds(0, SPLIT_A)], gsems[b])
        pltpu.async_copy(table.at[idx_v.at[pl.ds(off + SPLIT_A, SPLIT_B)]],
                         bufs[b].at[pl.ds(SPLIT_A, SPLIT_B)], gsems[b])

    def wait_gather(b):
        pltpu.make_async_copy(table.at[pl.ds(0, CHUNK)], bufs[b],
                              gsems[b]).wait()

    def compute(b):
        buf = bufs[b]

        def row(r, carry):
            for k in range(DIM // LANES):
                sl = pl.ds(k * LANES, LANES)
                buf[r, sl] = buf[r, sl] * SCALE + pe_v[r, sl]
            return carry

        lax.fori_loop(0, CHUNK, row, 0)

    def start_out(c, b):
        off = base + c * CHUNK
        pltpu.async_copy(bufs[b], out.at[pl.ds(off, CHUNK)], osems[b])

    def wait_out(b):
        pltpu.make_async_copy(bufs[b], out.at[pl.ds(0, CHUNK)],
                              osems[b]).wait()

    def step(c, b, *, first_use_next=False, start_next=True):
        wait_gather(b)
        compute(b)
        start_out(c, b)
        if start_next:
            bn = (b + 2) % NBUF
            if not first_use_next:
                wait_out(bn)
            start_gather(c + 2, bn)

    start_gather(0, 0)
    start_gather(1, 1)
    pltpu.make_async_copy(pe, pe_v, psem).wait()

    step(0, 0, first_use_next=True)

    def main(i, carry):
        for j in range(NBUF):
            c = 1 + i * NBUF + j
            step(c, (1 + j) % NBUF)
        return carry

    lax.fori_loop(0, 9, main, 0)

    step(28, (28 % NBUF))
    step(29, (29 % NBUF))
    step(30, (30 % NBUF), start_next=False)
    step(31, (31 % NBUF), start_next=False)

    for b in range(NBUF):
        wait_out(b)


def kernel(encoded_words, embed_weight):
    idx = encoded_words.reshape(-1).astype(jnp.int32)
    pe = _pos_encoding()
    out = _sc_embed(embed_weight, idx, pe)
    return out.reshape(encoded_words.shape[0], encoded_words.shape[1], DIM)
